# baseline (speedup 1.0000x reference)
import functools

import jax
import jax.numpy as jnp
from jax import lax
from jax.experimental import pallas as pl
from jax.experimental.pallas import tpu as pltpu
from jax.experimental.pallas import tpu_sc as plsc

IN_F = 128
OUT_F = 128
ALPHA = 0.2
N_NODES = 10000
N_EDGES = 320000

NC = 2
NS = 16
NW = NC * NS
EPW = N_EDGES // NW
CHUNK = 128
NCH = 80
PAD = NCH * CHUNK - EPW
GROUPS = CHUNK // 16
RSTRIPE = 624
RTAIL = N_NODES - NS * RSTRIPE

BLK = 1000


def _mm_body(x_ref, w_ref, at_ref, wh_ref, s_ref, c_ref):
    wh = jnp.dot(x_ref[...], w_ref[...], preferred_element_type=jnp.float32)
    wh_ref[...] = wh
    s = lax.dot_general(at_ref[...], wh, (((1,), (1,)), ((), ())),
                        preferred_element_type=jnp.float32)
    s_ref[...] = s
    m = jnp.max(s[0]) + jnp.max(s[1])
    c = jnp.where(m >= 0, m, ALPHA * m)
    c_ref[...] = jnp.full((8, 128), c, jnp.float32)


def _matmul_scores(x, W, At):
    return pl.pallas_call(
        _mm_body,
        out_shape=[
            jax.ShapeDtypeStruct((N_NODES, OUT_F), jnp.float32),
            jax.ShapeDtypeStruct((IN_F, N_NODES), jnp.float32),
            jax.ShapeDtypeStruct((8, 128), jnp.float32),
        ],
    )(x, W, At)


def _make_sc_kernel():
    mesh = plsc.VectorSubcoreMesh(core_axis_name="c", subcore_axis_name="s",
                                  num_cores=NC, num_subcores=NS)

    nbuf_scratch = []
    for _ in range(3):
        nbuf_scratch += [
            pltpu.VMEM((2 * CHUNK,), jnp.int32),
            pltpu.VMEM((CHUNK,), jnp.int32),
            pltpu.VMEM((CHUNK,), jnp.float32),
            pltpu.VMEM((CHUNK, OUT_F), jnp.float32),
        ]

    @functools.partial(
        pl.kernel,
        out_type=[
            jax.ShapeDtypeStruct((NC, N_NODES, OUT_F), jnp.float32),
            jax.ShapeDtypeStruct((NW, 1, 16), jnp.float32),
        ],
        mesh=mesh,
        scratch_types=nbuf_scratch + [
            pltpu.VMEM_SHARED((N_NODES + 8, OUT_F), jnp.float32),
            pltpu.SemaphoreType.DMA,
            pltpu.SemaphoreType.DMA,
            pltpu.SemaphoreType.DMA,
            pltpu.SemaphoreType.DMA,
            pltpu.SemaphoreType.DMA,
            pltpu.SemaphoreType.DMA,
            pltpu.SemaphoreType.DMA,
            pltpu.SemaphoreType.DMA,
            pltpu.SemaphoreType.DMA,
        ],
        compiler_params=pltpu.CompilerParams(needs_layout_passes=False),
    )
    def sc_kernel(wh_hbm, s1_hbm, s2_hbm, eint_hbm, c_hbm,
                  parts_hbm, psums_hbm,
                  eidx_a, sv1_a, sv2_a, rows_a,
                  eidx_b, sv1_b, sv2_b, rows_b,
                  eidx_c, sv1_c, sv2_c, rows_c,
                  acc,
                  semi_a, semi_b, semi_c,
                  semg_a, semg_b, semg_c,
                  sems_a, sems_b, sems_c):
        EIDX = [eidx_a, eidx_b, eidx_c]
        SV1 = [sv1_a, sv1_b, sv1_c]
        SV2 = [sv2_a, sv2_b, sv2_c]
        ROWS = [rows_a, rows_b, rows_c]
        SEMI = [semi_a, semi_b, semi_c]
        SEMG = [semg_a, semg_b, semg_c]
        SEMS = [sems_a, sems_b, sems_c]

        cid = lax.axis_index("c")
        sid = lax.axis_index("s")
        wid = sid * NC + cid
        ebase = wid * NCH * 2 * CHUNK

        zero16 = jnp.zeros((16,), jnp.float32)

        pltpu.sync_copy(c_hbm.at[0, pl.ds(0, 16)],
                        rows_a.at[0, pl.ds(0, 16)])
        c_shift = rows_a[0, pl.ds(0, 16)][0]

        def zrow(i, carry):
            for j in range(GROUPS):
                rows_a[i, pl.ds(j * 16, 16)] = zero16
            return carry

        lax.fori_loop(0, CHUNK, zrow, 0)
        zbase = sid * RSTRIPE
        for kk in range(RSTRIPE // CHUNK):
            pltpu.sync_copy(rows_a,
                            acc.at[pl.ds(zbase + kk * CHUNK, CHUNK)])
        if RSTRIPE % CHUNK:
            pltpu.sync_copy(
                rows_a.at[pl.ds(0, RSTRIPE % CHUNK)],
                acc.at[pl.ds(zbase + (RSTRIPE // CHUNK) * CHUNK,
                             RSTRIPE % CHUNK)])

        @pl.when(sid == NS - 1)
        def _zero_tail():
            pltpu.sync_copy(rows_a.at[pl.ds(0, RTAIL)],
                            acc.at[pl.ds(NS * RSTRIPE, RTAIL)])

        plsc.subcore_barrier()

        def load_idx(cc, b):
            base = ebase + cc * (2 * CHUNK)
            pltpu.async_copy(eint_hbm.at[pl.ds(base, 2 * CHUNK)], EIDX[b],
                             SEMI[b])

        def wait_idx(cc, b):
            base = ebase + cc * (2 * CHUNK)
            pltpu.make_async_copy(
                eint_hbm.at[pl.ds(base, 2 * CHUNK)], EIDX[b],
                SEMI[b]).wait()

        def issue_gathers(b):
            src_ref = EIDX[b].at[pl.ds(0, CHUNK)]
            dst_ref = EIDX[b].at[pl.ds(CHUNK, CHUNK)]
            pltpu.async_copy(wh_hbm.at[dst_ref], ROWS[b], SEMG[b])
            pltpu.async_copy(s1_hbm.at[src_ref], SV1[b], SEMG[b])
            pltpu.async_copy(s2_hbm.at[dst_ref], SV2[b], SEMG[b])

        def wait_gathers(b):
            src_ref = EIDX[b].at[pl.ds(0, CHUNK)]
            dst_ref = EIDX[b].at[pl.ds(CHUNK, CHUNK)]
            pltpu.make_async_copy(wh_hbm.at[dst_ref], ROWS[b],
                                  SEMG[b]).wait()
            pltpu.make_async_copy(s1_hbm.at[src_ref], SV1[b],
                                  SEMG[b]).wait()
            pltpu.make_async_copy(s2_hbm.at[dst_ref], SV2[b],
                                  SEMG[b]).wait()

        def drain_scatter(b):
            pltpu.make_async_copy(ROWS[b], acc.at[SV1[b]], SEMS[b]).wait()

        def weigh_scale(b, psum):
            sv1, sv2, rows = SV1[b], SV2[b], ROWS[b]

            def group(g, acc_p):
                s1v = plsc.bitcast(sv1[pl.ds(g * 16, 16)], jnp.float32)
                e = s1v + sv2[pl.ds(g * 16, 16)]
                e = jnp.where(e >= 0, e, ALPHA * e)
                p16 = jnp.exp(e - c_shift)
                for j in range(16):
                    pe = p16[j]
                    ei = g * 16 + j
                    for k in range(GROUPS):
                        sl = pl.ds(k * 16, 16)
                        rows[ei, sl] = rows[ei, sl] * pe
                return acc_p + p16

            return lax.fori_loop(0, GROUPS, group, psum)

        def copy_idx(b):
            for g in range(GROUPS):
                SV1[b][pl.ds(g * 16, 16)] = EIDX[b][pl.ds(g * 16, 16)]

        def phase(cc, b, psum, drain=True, lead=True, load=True):
            bn = (b + 1) % 3
            if lead:
                wait_idx(cc + 1, bn)
            if drain:
                drain_scatter(bn)
            if lead:
                issue_gathers(bn)
            wait_gathers(b)
            psum = weigh_scale(b, psum)
            copy_idx(b)
            pltpu.async_copy(ROWS[b], acc.at[SV1[b]], SEMS[b], add=True)
            if load:
                load_idx(cc + 3, b)
            return psum

        load_idx(0, 0)
        load_idx(1, 1)
        load_idx(2, 2)
        wait_idx(0, 0)
        issue_gathers(0)

        psum = jnp.zeros((16,), jnp.float32)
        psum = phase(0, 0, psum, drain=False)
        psum = phase(1, 1, psum, drain=False)
        psum = phase(2, 2, psum)

        def triple(j, ps):
            c0 = 3 * j
            ps = phase(c0, 0, ps)
            ps = phase(c0 + 1, 1, ps)
            ps = phase(c0 + 2, 2, ps)
            return ps

        psum = lax.fori_loop(1, (NCH - 5) // 3, triple, psum)

        psum = phase(NCH - 5, 0, psum)
        psum = phase(NCH - 4, 1, psum)
        psum = phase(NCH - 3, 2, psum, load=False)
        psum = phase(NCH - 2, 0, psum, load=False)
        psum = phase(NCH - 1, 1, psum, lead=False, load=False)
        drain_scatter(0)
        drain_scatter(1)

        plsc.subcore_barrier()

        pltpu.sync_copy(acc.at[pl.ds(sid * RSTRIPE, RSTRIPE)],
                        parts_hbm.at[cid, pl.ds(sid * RSTRIPE, RSTRIPE)])

        @pl.when(sid == NS - 1)
        def _out_tail():
            pltpu.sync_copy(acc.at[pl.ds(NS * RSTRIPE, RTAIL)],
                            parts_hbm.at[cid, pl.ds(NS * RSTRIPE, RTAIL)])

        rows_a[0, pl.ds(0, 16)] = psum
        pltpu.sync_copy(rows_a.at[0, pl.ds(0, 16)], psums_hbm.at[wid, 0])

    return sc_kernel


_sc_cache = None


def _get_sc_kernel():
    global _sc_cache
    if _sc_cache is None:
        _sc_cache = _make_sc_kernel()
    return _sc_cache


def _ep_body(parts_ref, psums_ref, o_ref):
    s = jnp.sum(psums_ref[...])
    v = (parts_ref[0] + parts_ref[1]) * (1.0 / s)
    o_ref[...] = jnp.where(v > 0, v, jnp.exp(v) - 1.0)


def _epilogue(parts, psums):
    return pl.pallas_call(
        _ep_body,
        grid=(N_NODES // BLK,),
        in_specs=[
            pl.BlockSpec((NC, BLK, OUT_F), lambda i: (0, i, 0)),
            pl.BlockSpec((NW, 1, 16), lambda i: (0, 0, 0)),
        ],
        out_specs=pl.BlockSpec((BLK, OUT_F), lambda i: (i, 0)),
        out_shape=jax.ShapeDtypeStruct((N_NODES, OUT_F), jnp.float32),
    )(parts, psums)


def kernel(x, edge_index, W, a):
    x = x.astype(jnp.float32)
    ei = edge_index.astype(jnp.int32)
    a_col = a[:, 0].astype(jnp.float32)
    At = jnp.zeros((IN_F, IN_F), jnp.float32)
    At = At.at[0, :].set(a_col[:IN_F]).at[1, :].set(a_col[IN_F:])
    Wh, s_all, c_arr = _matmul_scores(x, W.astype(jnp.float32), At)
    s1_ext = jnp.concatenate(
        [s_all[0], jnp.full((8,), -1e30, jnp.float32)])
    s1_bits = lax.bitcast_convert_type(s1_ext, jnp.int32)
    src = ei[0].reshape(NW, EPW)
    dst = ei[1].reshape(NW, EPW)
    src = jnp.pad(src, ((0, 0), (0, PAD)), constant_values=N_NODES)
    dst = jnp.pad(dst, ((0, 0), (0, PAD)), constant_values=0)
    eint = jnp.stack([src.reshape(NW, NCH, CHUNK),
                      dst.reshape(NW, NCH, CHUNK)], axis=2).reshape(-1)
    parts, psums = _get_sc_kernel()(Wh, s1_bits, s_all[1], eint, c_arr)
    return _epilogue(parts, psums)


# speedup vs baseline: 2.5689x; 2.5689x over previous
import functools

import jax
import jax.numpy as jnp
from jax import lax
from jax.experimental import pallas as pl
from jax.experimental.pallas import tpu as pltpu
from jax.experimental.pallas import tpu_sc as plsc

IN_F = 128
OUT_F = 128
ALPHA = 0.2
N_NODES = 10000
N_EDGES = 320000

NC = 2
NS = 16
NW = NC * NS
EPW = N_EDGES // NW
CHUNK = 128
NFULL = EPW // CHUNK
TAIL = EPW - NFULL * CHUNK
GROUPS = CHUNK // 16
RSTRIPE = 624
RTAIL = N_NODES - NS * RSTRIPE

BLK = 1000


def _mm_body(x_ref, w_ref, at_ref, wh_ref, s_ref, c_ref):
    wh = jnp.dot(x_ref[...], w_ref[...], preferred_element_type=jnp.float32)
    wh_ref[...] = wh
    s = lax.dot_general(at_ref[...], wh, (((1,), (1,)), ((), ())),
                        preferred_element_type=jnp.float32)
    s_ref[...] = s
    m = jnp.max(s[0]) + jnp.max(s[1])
    c = jnp.where(m >= 0, m, ALPHA * m)
    c_ref[...] = jnp.full((8, 128), c, jnp.float32)


def _matmul_scores(x, W, At):
    return pl.pallas_call(
        _mm_body,
        out_shape=[
            jax.ShapeDtypeStruct((N_NODES, OUT_F), jnp.float32),
            jax.ShapeDtypeStruct((IN_F, N_NODES), jnp.float32),
            jax.ShapeDtypeStruct((8, 128), jnp.float32),
        ],
    )(x, W, At)


def _make_sc_kernel():
    mesh = plsc.VectorSubcoreMesh(core_axis_name="c", subcore_axis_name="s",
                                  num_cores=NC, num_subcores=NS)

    nbuf_scratch = []
    for _ in range(3):
        nbuf_scratch += [
            pltpu.VMEM((CHUNK,), jnp.int32),
            pltpu.VMEM((CHUNK,), jnp.int32),
            pltpu.VMEM((CHUNK,), jnp.int32),
            pltpu.VMEM((CHUNK,), jnp.float32),
            pltpu.VMEM((CHUNK, OUT_F), jnp.float32),
        ]

    @functools.partial(
        pl.kernel,
        out_type=[
            jax.ShapeDtypeStruct((NC, N_NODES, OUT_F), jnp.float32),
            jax.ShapeDtypeStruct((NW, 1, 16), jnp.float32),
        ],
        mesh=mesh,
        scratch_types=nbuf_scratch + [
            pltpu.VMEM_SHARED((N_NODES, OUT_F), jnp.float32),
            pltpu.SemaphoreType.DMA,
            pltpu.SemaphoreType.DMA,
            pltpu.SemaphoreType.DMA,
            pltpu.SemaphoreType.DMA,
            pltpu.SemaphoreType.DMA,
            pltpu.SemaphoreType.DMA,
            pltpu.SemaphoreType.DMA,
            pltpu.SemaphoreType.DMA,
            pltpu.SemaphoreType.DMA,
        ],
        compiler_params=pltpu.CompilerParams(needs_layout_passes=False),
    )
    def sc_kernel(wh_hbm, s1_hbm, s2_hbm, esrc_hbm, edst_hbm, c_hbm,
                  parts_hbm, psums_hbm,
                  sidx_a, didx_a, sv1_a, sv2_a, rows_a,
                  sidx_b, didx_b, sv1_b, sv2_b, rows_b,
                  sidx_c, didx_c, sv1_c, sv2_c, rows_c,
                  acc,
                  semi_a, semi_b, semi_c,
                  semg_a, semg_b, semg_c,
                  sems_a, sems_b, sems_c):
        SIDX = [sidx_a, sidx_b, sidx_c]
        DIDX = [didx_a, didx_b, didx_c]
        SV1 = [sv1_a, sv1_b, sv1_c]
        SV2 = [sv2_a, sv2_b, sv2_c]
        ROWS = [rows_a, rows_b, rows_c]
        SEMI = [semi_a, semi_b, semi_c]
        SEMG = [semg_a, semg_b, semg_c]
        SEMS = [sems_a, sems_b, sems_c]

        cid = lax.axis_index("c")
        sid = lax.axis_index("s")
        wid = sid * NC + cid
        ebase = wid * EPW

        zero16 = jnp.zeros((16,), jnp.float32)

        pltpu.sync_copy(c_hbm.at[0, pl.ds(0, 16)],
                        rows_a.at[0, pl.ds(0, 16)])
        c_shift = rows_a[0, pl.ds(0, 16)][0]

        def zrow(i, carry):
            for j in range(GROUPS):
                rows_a[i, pl.ds(j * 16, 16)] = zero16
            return carry

        lax.fori_loop(0, CHUNK, zrow, 0)
        zbase = sid * RSTRIPE
        for kk in range(RSTRIPE // CHUNK):
            pltpu.sync_copy(rows_a,
                            acc.at[pl.ds(zbase + kk * CHUNK, CHUNK)])
        if RSTRIPE % CHUNK:
            pltpu.sync_copy(
                rows_a.at[pl.ds(0, RSTRIPE % CHUNK)],
                acc.at[pl.ds(zbase + (RSTRIPE // CHUNK) * CHUNK,
                             RSTRIPE % CHUNK)])

        @pl.when(sid == NS - 1)
        def _zero_tail():
            pltpu.sync_copy(rows_a.at[pl.ds(0, RTAIL)],
                            acc.at[pl.ds(NS * RSTRIPE, RTAIL)])

        plsc.subcore_barrier()

        def load_idx(cc, b):
            base = ebase + cc * CHUNK
            pltpu.async_copy(esrc_hbm.at[pl.ds(base, CHUNK)], SIDX[b],
                             SEMI[b])
            pltpu.async_copy(edst_hbm.at[pl.ds(base, CHUNK)], DIDX[b],
                             SEMI[b])

        def wait_idx(cc, b):
            base = ebase + cc * CHUNK
            pltpu.make_async_copy(
                esrc_hbm.at[pl.ds(base, CHUNK)], SIDX[b], SEMI[b]).wait()
            pltpu.make_async_copy(
                edst_hbm.at[pl.ds(base, CHUNK)], DIDX[b], SEMI[b]).wait()

        def issue_gathers(b):
            pltpu.async_copy(wh_hbm.at[DIDX[b]], ROWS[b], SEMG[b])
            pltpu.async_copy(s1_hbm.at[SIDX[b]], SV1[b], SEMG[b])
            pltpu.async_copy(s2_hbm.at[DIDX[b]], SV2[b], SEMG[b])

        def wait_gathers(b):
            pltpu.make_async_copy(wh_hbm.at[DIDX[b]], ROWS[b],
                                  SEMG[b]).wait()
            pltpu.make_async_copy(s1_hbm.at[SIDX[b]], SV1[b],
                                  SEMG[b]).wait()
            pltpu.make_async_copy(s2_hbm.at[DIDX[b]], SV2[b],
                                  SEMG[b]).wait()

        def drain_scatter(b):
            pltpu.make_async_copy(ROWS[b], acc.at[SV1[b]], SEMS[b]).wait()

        def weigh_scale(b, psum):
            sv1, sv2, rows = SV1[b], SV2[b], ROWS[b]

            def group(g, acc_p):
                s1v = plsc.bitcast(sv1[pl.ds(g * 16, 16)], jnp.float32)
                e = s1v + sv2[pl.ds(g * 16, 16)]
                e = jnp.where(e >= 0, e, ALPHA * e)
                p16 = jnp.exp(e - c_shift)
                for j in range(16):
                    pe = p16[j]
                    ei = g * 16 + j
                    for k in range(GROUPS):
                        sl = pl.ds(k * 16, 16)
                        rows[ei, sl] = rows[ei, sl] * pe
                return acc_p + p16

            return lax.fori_loop(0, GROUPS, group, psum)

        def copy_idx(b):
            for g in range(GROUPS):
                SV1[b][pl.ds(g * 16, 16)] = SIDX[b][pl.ds(g * 16, 16)]

        def phase(cc, b, psum, drain=True, lead=True, load=True):
            bn = (b + 1) % 3
            if lead:
                wait_idx(cc + 1, bn)
            if drain:
                drain_scatter(bn)
            if lead:
                issue_gathers(bn)
            wait_gathers(b)
            psum = weigh_scale(b, psum)
            copy_idx(b)
            pltpu.async_copy(ROWS[b], acc.at[SV1[b]], SEMS[b], add=True)
            if load:
                load_idx(cc + 3, b)
            return psum

        load_idx(0, 0)
        load_idx(1, 1)
        load_idx(2, 2)
        wait_idx(0, 0)
        issue_gathers(0)

        psum = jnp.zeros((16,), jnp.float32)
        psum = phase(0, 0, psum, drain=False)
        psum = phase(1, 1, psum, drain=False)
        psum = phase(2, 2, psum)

        def triple(j, ps):
            c0 = 3 * j
            ps = phase(c0, 0, ps)
            ps = phase(c0 + 1, 1, ps)
            ps = phase(c0 + 2, 2, ps)
            return ps

        psum = lax.fori_loop(1, NFULL // 3 - 1, triple, psum)

        psum = phase(NFULL - 3, 0, psum, load=False)
        psum = phase(NFULL - 2, 1, psum, load=False)
        psum = phase(NFULL - 1, 2, psum, lead=False, load=False)

        tbase = ebase + NFULL * CHUNK
        pltpu.sync_copy(esrc_hbm.at[pl.ds(tbase, TAIL)],
                        sidx_a.at[pl.ds(0, TAIL)])
        pltpu.sync_copy(edst_hbm.at[pl.ds(tbase, TAIL)],
                        didx_a.at[pl.ds(0, TAIL)])
        issue_gathers(0)
        wait_gathers(0)
        e = (plsc.bitcast(sv1_a[pl.ds(0, 16)], jnp.float32)
             + sv2_a[pl.ds(0, 16)])
        e = jnp.where(e >= 0, e, ALPHA * e)
        tp = jnp.exp(e - c_shift)
        psum = psum + tp
        for j in range(16):
            pe = tp[j]
            for k in range(GROUPS):
                sl = pl.ds(k * 16, 16)
                rows_a[j, sl] = rows_a[j, sl] * pe

        def zero_rest(i, carry):
            for k in range(GROUPS):
                rows_a[i, pl.ds(k * 16, 16)] = zero16
            return carry

        lax.fori_loop(TAIL, CHUNK, zero_rest, 0)
        copy_idx(0)
        pltpu.sync_copy(rows_a, acc.at[sv1_a], add=True)
        drain_scatter(1)
        drain_scatter(2)

        plsc.subcore_barrier()

        pltpu.sync_copy(acc.at[pl.ds(sid * RSTRIPE, RSTRIPE)],
                        parts_hbm.at[cid, pl.ds(sid * RSTRIPE, RSTRIPE)])

        @pl.when(sid == NS - 1)
        def _out_tail():
            pltpu.sync_copy(acc.at[pl.ds(NS * RSTRIPE, RTAIL)],
                            parts_hbm.at[cid, pl.ds(NS * RSTRIPE, RTAIL)])

        rows_a[0, pl.ds(0, 16)] = psum
        pltpu.sync_copy(rows_a.at[0, pl.ds(0, 16)], psums_hbm.at[wid, 0])

    return sc_kernel


_sc_cache = None


def _get_sc_kernel():
    global _sc_cache
    if _sc_cache is None:
        _sc_cache = _make_sc_kernel()
    return _sc_cache


def _ep_body(parts_ref, psums_ref, o_ref):
    s = jnp.sum(psums_ref[...])
    v = (parts_ref[0] + parts_ref[1]) * (1.0 / s)
    o_ref[...] = jnp.where(v > 0, v, jnp.exp(v) - 1.0)


def _epilogue(parts, psums):
    return pl.pallas_call(
        _ep_body,
        grid=(N_NODES // BLK,),
        in_specs=[
            pl.BlockSpec((NC, BLK, OUT_F), lambda i: (0, i, 0)),
            pl.BlockSpec((NW, 1, 16), lambda i: (0, 0, 0)),
        ],
        out_specs=pl.BlockSpec((BLK, OUT_F), lambda i: (i, 0)),
        out_shape=jax.ShapeDtypeStruct((N_NODES, OUT_F), jnp.float32),
    )(parts, psums)


def kernel(x, edge_index, W, a):
    x = x.astype(jnp.float32)
    ei = edge_index.astype(jnp.int32)
    a_col = a[:, 0].astype(jnp.float32)
    At = jnp.zeros((IN_F, IN_F), jnp.float32)
    At = At.at[0, :].set(a_col[:IN_F]).at[1, :].set(a_col[IN_F:])
    Wh, s_all, c_arr = _matmul_scores(x, W.astype(jnp.float32), At)
    s1_bits = lax.bitcast_convert_type(s_all[0], jnp.int32)
    parts, psums = _get_sc_kernel()(Wh, s1_bits, s_all[1],
                                    ei[0], ei[1], c_arr)
    return _epilogue(parts, psums)
